# baseline (speedup 1.0000x reference)
import functools

import jax
import jax.numpy as jnp
from jax import lax
from jax.experimental import pallas as pl
from jax.experimental.pallas import tpu as pltpu
from jax.experimental.pallas import tpu_sc as plsc

_NUM_CODES = 8192
_DIM = 32
_TOKENS = 2 * 16 * 32 * 32
_T = 1024
_K = 2048


def _argmin_body(x_ref, dt2_ref, out_ref, dn_ref, w_ref, li_ref):
    @pl.when(pl.program_id(0) == 0)
    def _():
        d2 = dt2_ref[...]
        dn_ref[...] = 0.25 * jnp.sum(d2 * d2, axis=0, keepdims=True)
        col = lax.broadcasted_iota(jnp.int32, (_K, 128), 1)
        row = lax.broadcasted_iota(jnp.int32, (_K, 128), 0)
        hi = (row // 256).astype(jnp.float32)
        lo = (row % 256).astype(jnp.float32)
        wf = jnp.where(col == 0, hi,
                       jnp.where(col == 1, lo,
                                 jnp.where(col == 2, 1.0, 0.0)))
        w_ref[...] = wf.astype(jnp.bfloat16)

    x = x_ref[...]
    tn = jnp.sum(x * x, axis=1, keepdims=True)
    big = jnp.float32(_NUM_CODES)
    best_v = jnp.full((_T, 1), jnp.inf, jnp.float32)
    best_i = jnp.zeros((_T, 1), jnp.float32)
    for c in range(_NUM_CODES // _K):
        dots2 = lax.dot_general(
            x, dt2_ref[:, c * _K:(c + 1) * _K],
            (((1,), (0,)), ((), ())),
            preferred_element_type=jnp.float32,
        )
        v = (dots2 + dn_ref[:, c * _K:(c + 1) * _K]) + tn
        m = jnp.min(v, axis=1, keepdims=True)
        eq16 = jnp.where(v == m, 1.0, 0.0).astype(jnp.bfloat16)
        s = lax.dot_general(eq16, w_ref[...], (((1,), (0,)), ((), ())),
                            preferred_element_type=jnp.float32)
        li_ref[...] = s[:, 0:1] * 256.0 + s[:, 1:2]
        cnt = s[:, 2:3]

        @pl.when(jnp.max(cnt) > 1.5)
        def _():
            iota = lax.broadcasted_iota(
                jnp.int32, (_T, _K), 1).astype(jnp.float32)
            li_ref[...] = jnp.min(jnp.where(v == m, iota, big),
                                  axis=1, keepdims=True)

        li = li_ref[...]
        upd = m < best_v
        best_i = jnp.where(upd, li + jnp.float32(c * _K), best_i)
        best_v = jnp.where(upd, m, best_v)
    out_ref[...] = best_i.astype(jnp.int32)


def _argmin_call(x_all, dt2):
    ntok = x_all.shape[0]
    return pl.pallas_call(
        _argmin_body,
        grid=(ntok // _T,),
        in_specs=[
            pl.BlockSpec((_T, _DIM), lambda i: (i, 0)),
            pl.BlockSpec((_DIM, _NUM_CODES), lambda i: (0, 0)),
        ],
        out_specs=pl.BlockSpec((_T, 1), lambda i: (i, 0)),
        out_shape=jax.ShapeDtypeStruct((ntok, 1), jnp.int32),
        scratch_shapes=[pltpu.VMEM((1, _NUM_CODES), jnp.float32),
                        pltpu.VMEM((_K, 128), jnp.bfloat16),
                        pltpu.VMEM((_T, 1), jnp.float32)],
    )(x_all, dt2)


def _make_gather(ntok):
    info = plsc.get_sparse_core_info()
    nc, ns = info.num_cores, info.num_subcores
    nw = nc * ns
    b_per_w = ntok // nw
    nchunk = b_per_w // 128
    mesh = plsc.VectorSubcoreMesh(core_axis_name="c", subcore_axis_name="s")

    @functools.partial(
        pl.kernel, mesh=mesh,
        out_type=jax.ShapeDtypeStruct((ntok, 128), jnp.float32),
        scratch_types=[
            pltpu.VMEM((nchunk, 128), jnp.int32),
            pltpu.VMEM((2, 128, 128), jnp.float32),
            pltpu.SemaphoreType.DMA,
            pltpu.SemaphoreType.DMA,
        ],
    )
    def gather_k(table_hbm, idx_hbm, out_hbm, idx_v, buf, sem0, sem1):
        wid = lax.axis_index("s") * nc + lax.axis_index("c")
        base = wid * b_per_w
        pltpu.sync_copy(idx_hbm.at[wid], idx_v)
        sems = [sem0, sem1]
        cps = [None, None]
        for j in range(nchunk):
            b = j % 2
            cps[b] = pltpu.async_copy(table_hbm.at[idx_v.at[j]], buf.at[b],
                                      sems[b])
            if j >= 1:
                pb = (j - 1) % 2
                cps[pb].wait()
                pltpu.sync_copy(buf.at[pb],
                                out_hbm.at[pl.ds(base + (j - 1) * 128, 128)])
        lb = (nchunk - 1) % 2
        cps[lb].wait()
        pltpu.sync_copy(buf.at[lb],
                        out_hbm.at[pl.ds(base + (nchunk - 1) * 128, 128)])

    return gather_k, nw


def kernel(inputs, inputs_thermal, dictionary):
    n, h, w = inputs.shape[0], inputs.shape[2], inputs.shape[3]
    ntok = n * h * w
    dt2 = dictionary.T * jnp.float32(-2.0)
    table128 = jnp.pad(dictionary, ((0, 0), (0, 128 - _DIM)))
    gather_k, nw = _make_gather(ntok)

    def one_stream(inp):
        x = jnp.transpose(inp, (0, 2, 3, 1)).reshape(-1, _DIM)
        idx = _argmin_call(x, dt2).reshape(ntok)
        emb = gather_k(table128, idx.reshape(nw, -1, 128))[:, :_DIM]
        e = emb.reshape(n, h, w, _DIM).transpose(0, 3, 1, 2)
        return e, idx.reshape(n, h, w)

    e1, i1 = one_stream(inputs)
    e2, i2 = one_stream(inputs_thermal)
    return (e1, e1, i1, e2, e2, i2)


# speedup vs baseline: 1.9631x; 1.9631x over previous
import functools

import jax
import jax.numpy as jnp
from jax import lax
from jax.experimental import pallas as pl
from jax.experimental.pallas import tpu as pltpu
from jax.experimental.pallas import tpu_sc as plsc

_NUM_CODES = 8192
_DIM = 32
_TOKENS = 2 * 16 * 32 * 32
_T = 1024
_K = 2048


def _argmin_body(x_ref, dt2_ref, out_ref, dn_ref):
    @pl.when(pl.program_id(0) == 0)
    def _():
        d2 = dt2_ref[...]
        dn_ref[...] = 0.25 * jnp.sum(d2 * d2, axis=0, keepdims=True)

    x = x_ref[...]
    tn = jnp.sum(x * x, axis=1, keepdims=True)
    iota = lax.broadcasted_iota(jnp.int32, (_T, _K), 1).astype(jnp.float32)
    big = jnp.float32(_NUM_CODES)
    best_v = jnp.full((_T, 1), jnp.inf, jnp.float32)
    best_i = jnp.zeros((_T, 1), jnp.float32)
    for c in range(_NUM_CODES // _K):
        dots2 = lax.dot_general(
            x, dt2_ref[:, c * _K:(c + 1) * _K],
            (((1,), (0,)), ((), ())),
            preferred_element_type=jnp.float32,
        )
        v = (dots2 + dn_ref[:, c * _K:(c + 1) * _K]) + tn
        m = jnp.min(v, axis=1, keepdims=True)
        li = jnp.min(jnp.where(v == m, iota, big), axis=1, keepdims=True)
        upd = m < best_v
        best_i = jnp.where(upd, li + jnp.float32(c * _K), best_i)
        best_v = jnp.where(upd, m, best_v)
    out_ref[...] = best_i.astype(jnp.int32)


def _argmin_call(x_all, dt2):
    ntok = x_all.shape[0]
    return pl.pallas_call(
        _argmin_body,
        grid=(ntok // _T,),
        in_specs=[
            pl.BlockSpec((_T, _DIM), lambda i: (i, 0)),
            pl.BlockSpec((_DIM, _NUM_CODES), lambda i: (0, 0)),
        ],
        out_specs=pl.BlockSpec((_T, 1), lambda i: (i, 0)),
        out_shape=jax.ShapeDtypeStruct((ntok, 1), jnp.int32),
        scratch_shapes=[pltpu.VMEM((1, _NUM_CODES), jnp.float32)],
    )(x_all, dt2)


def _make_gather(ntok):
    info = plsc.get_sparse_core_info()
    nc, ns = info.num_cores, info.num_subcores
    nw = nc * ns
    b_per_w = ntok // nw
    nchunk = b_per_w // 128
    mesh = plsc.VectorSubcoreMesh(core_axis_name="c", subcore_axis_name="s")

    @functools.partial(
        pl.kernel, mesh=mesh,
        out_type=jax.ShapeDtypeStruct((ntok, 128), jnp.float32),
        scratch_types=[
            pltpu.VMEM((nchunk, 128), jnp.int32),
            pltpu.VMEM((2, 128, 128), jnp.float32),
            pltpu.SemaphoreType.DMA,
            pltpu.SemaphoreType.DMA,
        ],
    )
    def gather_k(table_hbm, idx_hbm, out_hbm, idx_v, buf, sem0, sem1):
        wid = lax.axis_index("s") * nc + lax.axis_index("c")
        base = wid * b_per_w
        pltpu.sync_copy(idx_hbm.at[wid], idx_v)
        sems = [sem0, sem1]
        cps = [None, None]
        for j in range(nchunk):
            b = j % 2
            cps[b] = pltpu.async_copy(table_hbm.at[idx_v.at[j]], buf.at[b],
                                      sems[b])
            if j >= 1:
                pb = (j - 1) % 2
                cps[pb].wait()
                pltpu.sync_copy(buf.at[pb],
                                out_hbm.at[pl.ds(base + (j - 1) * 128, 128)])
        lb = (nchunk - 1) % 2
        cps[lb].wait()
        pltpu.sync_copy(buf.at[lb],
                        out_hbm.at[pl.ds(base + (nchunk - 1) * 128, 128)])

    return gather_k, nw


def kernel(inputs, inputs_thermal, dictionary):
    n, h, w = inputs.shape[0], inputs.shape[2], inputs.shape[3]
    ntok = n * h * w
    dt2 = dictionary.T * jnp.float32(-2.0)
    table128 = jnp.pad(dictionary, ((0, 0), (0, 128 - _DIM)))
    gather_k, nw = _make_gather(ntok)

    def one_stream(inp):
        x = jnp.transpose(inp, (0, 2, 3, 1)).reshape(-1, _DIM)
        idx = _argmin_call(x, dt2).reshape(ntok)
        emb = gather_k(table128, idx.reshape(nw, -1, 128))[:, :_DIM]
        e = emb.reshape(n, h, w, _DIM).transpose(0, 3, 1, 2)
        return e, idx.reshape(n, h, w)

    e1, i1 = one_stream(inputs)
    e2, i2 = one_stream(inputs_thermal)
    return (e1, e1, i1, e2, e2, i2)
